# baseline (speedup 1.0000x reference)
import functools

import jax
import jax.numpy as jnp
from jax import lax
from jax.experimental import pallas as pl
from jax.experimental.pallas import tpu as pltpu
from jax.experimental.pallas import tpu_sc as plsc

D = 1024
L = 16
NW = 32
R_CHUNK = 16
NBUF = 3
BLOCK_ROWS = 1024


def _tc_body_one(mod_ref, x_ref, o_ref):
    o_ref[...] = x_ref[...] + mod_ref[1:2, :]


def _tc_add_one(x2, mod_emb):
    n = x2.shape[0]
    return pl.pallas_call(
        _tc_body_one,
        grid=(n // BLOCK_ROWS,),
        in_specs=[
            pl.BlockSpec((2, D), lambda i: (0, 0)),
            pl.BlockSpec((BLOCK_ROWS, D), lambda i: (i, 0)),
        ],
        out_specs=pl.BlockSpec((BLOCK_ROWS, D), lambda i: (i, 0)),
        out_shape=jax.ShapeDtypeStruct((n, D), jnp.float32),
    )(mod_emb, x2)


def _sc_compute_chunk(src, dst, m):
    for band in range(D // (16 * L)):
        mjs = [m[pl.ds((band * 16 + jj) * L, L)] for jj in range(16)]

        def row_body(r, _):
            base = band * 16 * L
            for jj in range(16):
                o = base + jj * L
                dst[r, pl.ds(o, L)] = src[r, pl.ds(o, L)] + mjs[jj]
            return 0

        lax.fori_loop(0, R_CHUNK, row_body, 0)


def _sc_add_one(x2, mod_emb, row):
    n = x2.shape[0]
    rows_w = n // NW
    chunks = rows_w // R_CHUNK
    full_iters = chunks // NBUF
    rem = chunks % NBUF
    assert full_iters >= 1
    mesh = plsc.VectorSubcoreMesh(core_axis_name="c", subcore_axis_name="s")

    @functools.partial(
        pl.kernel,
        mesh=mesh,
        out_type=jax.ShapeDtypeStruct((n, D), jnp.float32),
        scratch_types=(
            [pltpu.VMEM((R_CHUNK, D), jnp.float32)] * (2 * NBUF)
            + [pltpu.VMEM((D,), jnp.float32)]
            + [pltpu.SemaphoreType.DMA] * (2 * NBUF)
        ),
    )
    def k(x_hbm, mod_hbm, out_hbm, *scratch):
        ins = scratch[:NBUF]
        outs = scratch[NBUF:2 * NBUF]
        m0 = scratch[2 * NBUF]
        sins = scratch[2 * NBUF + 1:2 * NBUF + 1 + NBUF]
        souts = scratch[2 * NBUF + 1 + NBUF:]
        wid = lax.axis_index("s") * 2 + lax.axis_index("c")
        base = wid * rows_w
        pltpu.sync_copy(mod_hbm.at[row], m0)

        for b in range(NBUF):
            pltpu.async_copy(x_hbm.at[pl.ds(base + b * R_CHUNK, R_CHUNK)],
                             ins[b], sins[b])

        def do_chunk(c, b, maybe_first):
            off = base + c * R_CHUNK
            pltpu.make_async_copy(
                x_hbm.at[pl.ds(off, R_CHUNK)], ins[b], sins[b]).wait()

            def retire():
                pltpu.make_async_copy(
                    outs[b], out_hbm.at[pl.ds(off, R_CHUNK)], souts[b]).wait()
            if maybe_first is None:
                retire()
            else:
                pl.when(maybe_first > 0)(retire)

            _sc_compute_chunk(ins[b], outs[b], m0)
            def refill():
                pltpu.async_copy(
                    x_hbm.at[pl.ds(off + NBUF * R_CHUNK, R_CHUNK)],
                    ins[b], sins[b])
            if isinstance(c, int):
                if c + NBUF < chunks:
                    refill()
            else:
                pl.when(c + NBUF < chunks)(refill)
            pltpu.async_copy(outs[b], out_hbm.at[pl.ds(off, R_CHUNK)],
                             souts[b])

        def body(i, _):
            for b in range(NBUF):
                do_chunk(NBUF * i + b, b, i)
            return 0

        lax.fori_loop(0, full_iters, body, 0)
        for j in range(rem):
            do_chunk(NBUF * full_iters + j, j, None)
        for b in range(NBUF):
            pltpu.make_async_copy(
                outs[b], out_hbm.at[pl.ds(base, R_CHUNK)], souts[b]).wait()

    return k(x2, mod_emb)


def kernel(vision, language, mod_emb):
    b, lv, d = vision.shape
    _, lt, _ = language.shape
    ov = _sc_add_one(vision.reshape(b * lv, d), mod_emb, 0)
    ol = _tc_add_one(language.reshape(b * lt, d), mod_emb)
    return ov.reshape(b, lv, d), ol.reshape(b, lt, d)


# speedup vs baseline: 1.0089x; 1.0089x over previous
import functools

import jax
import jax.numpy as jnp
from jax import lax
from jax.experimental import pallas as pl
from jax.experimental.pallas import tpu as pltpu
from jax.experimental.pallas import tpu_sc as plsc

D = 1024
L = 16
NW = 32
R_CHUNK = 8
NBUF = 6
BLOCK_ROWS = 1024


def _tc_body_one(mod_ref, x_ref, o_ref):
    o_ref[...] = x_ref[...] + mod_ref[1:2, :]


def _tc_add_one(x2, mod_emb):
    n = x2.shape[0]
    return pl.pallas_call(
        _tc_body_one,
        grid=(n // BLOCK_ROWS,),
        in_specs=[
            pl.BlockSpec((2, D), lambda i: (0, 0)),
            pl.BlockSpec((BLOCK_ROWS, D), lambda i: (i, 0)),
        ],
        out_specs=pl.BlockSpec((BLOCK_ROWS, D), lambda i: (i, 0)),
        out_shape=jax.ShapeDtypeStruct((n, D), jnp.float32),
    )(mod_emb, x2)


def _sc_compute_chunk(src, dst, m):
    for band in range(D // (16 * L)):
        mjs = [m[pl.ds((band * 16 + jj) * L, L)] for jj in range(16)]

        def row_body(r, _):
            base = band * 16 * L
            for jj in range(16):
                o = base + jj * L
                dst[r, pl.ds(o, L)] = src[r, pl.ds(o, L)] + mjs[jj]
            return 0

        lax.fori_loop(0, R_CHUNK, row_body, 0)


def _sc_add_one(x2, mod_emb, row):
    n = x2.shape[0]
    rows_w = n // NW
    chunks = rows_w // R_CHUNK
    full_iters = chunks // NBUF
    rem = chunks % NBUF
    assert full_iters >= 1
    mesh = plsc.VectorSubcoreMesh(core_axis_name="c", subcore_axis_name="s")

    @functools.partial(
        pl.kernel,
        mesh=mesh,
        out_type=jax.ShapeDtypeStruct((n, D), jnp.float32),
        scratch_types=(
            [pltpu.VMEM((R_CHUNK, D), jnp.float32)] * (2 * NBUF)
            + [pltpu.VMEM((D,), jnp.float32)]
            + [pltpu.SemaphoreType.DMA] * (2 * NBUF)
        ),
    )
    def k(x_hbm, mod_hbm, out_hbm, *scratch):
        ins = scratch[:NBUF]
        outs = scratch[NBUF:2 * NBUF]
        m0 = scratch[2 * NBUF]
        sins = scratch[2 * NBUF + 1:2 * NBUF + 1 + NBUF]
        souts = scratch[2 * NBUF + 1 + NBUF:]
        wid = lax.axis_index("s") * 2 + lax.axis_index("c")
        base = wid * rows_w
        pltpu.sync_copy(mod_hbm.at[row], m0)

        for b in range(NBUF):
            pltpu.async_copy(x_hbm.at[pl.ds(base + b * R_CHUNK, R_CHUNK)],
                             ins[b], sins[b])

        def do_chunk(c, b, maybe_first):
            off = base + c * R_CHUNK
            pltpu.make_async_copy(
                x_hbm.at[pl.ds(off, R_CHUNK)], ins[b], sins[b]).wait()

            def retire():
                pltpu.make_async_copy(
                    outs[b], out_hbm.at[pl.ds(off, R_CHUNK)], souts[b]).wait()
            if maybe_first is None:
                retire()
            else:
                pl.when(maybe_first > 0)(retire)

            _sc_compute_chunk(ins[b], outs[b], m0)
            def refill():
                pltpu.async_copy(
                    x_hbm.at[pl.ds(off + NBUF * R_CHUNK, R_CHUNK)],
                    ins[b], sins[b])
            if isinstance(c, int):
                if c + NBUF < chunks:
                    refill()
            else:
                pl.when(c + NBUF < chunks)(refill)
            pltpu.async_copy(outs[b], out_hbm.at[pl.ds(off, R_CHUNK)],
                             souts[b])

        def body(i, _):
            for b in range(NBUF):
                do_chunk(NBUF * i + b, b, i)
            return 0

        lax.fori_loop(0, full_iters, body, 0)
        for j in range(rem):
            do_chunk(NBUF * full_iters + j, j, None)
        for b in range(NBUF):
            pltpu.make_async_copy(
                outs[b], out_hbm.at[pl.ds(base, R_CHUNK)], souts[b]).wait()

    return k(x2, mod_emb)


def kernel(vision, language, mod_emb):
    b, lv, d = vision.shape
    _, lt, _ = language.shape
    ov = _sc_add_one(vision.reshape(b * lv, d), mod_emb, 0)
    ol = _tc_add_one(language.reshape(b * lt, d), mod_emb)
    return ov.reshape(b, lv, d), ol.reshape(b, lt, d)


# speedup vs baseline: 1.0126x; 1.0037x over previous
import functools

import jax
import jax.numpy as jnp
from jax import lax
from jax.experimental import pallas as pl
from jax.experimental.pallas import tpu as pltpu
from jax.experimental.pallas import tpu_sc as plsc

D = 1024
L = 16
NW = 32
R_CHUNK = 8
NBUF = 4
BLOCK_ROWS = 1024


def _tc_body_one(mod_ref, x_ref, o_ref):
    o_ref[...] = x_ref[...] + mod_ref[1:2, :]


def _tc_add_one(x2, mod_emb):
    n = x2.shape[0]
    return pl.pallas_call(
        _tc_body_one,
        grid=(n // BLOCK_ROWS,),
        in_specs=[
            pl.BlockSpec((2, D), lambda i: (0, 0)),
            pl.BlockSpec((BLOCK_ROWS, D), lambda i: (i, 0)),
        ],
        out_specs=pl.BlockSpec((BLOCK_ROWS, D), lambda i: (i, 0)),
        out_shape=jax.ShapeDtypeStruct((n, D), jnp.float32),
    )(mod_emb, x2)


def _sc_compute_chunk(src, dst, m):
    for band in range(D // (16 * L)):
        mjs = [m[pl.ds((band * 16 + jj) * L, L)] for jj in range(16)]

        def row_body(r, _):
            base = band * 16 * L
            for jj in range(16):
                o = base + jj * L
                dst[r, pl.ds(o, L)] = src[r, pl.ds(o, L)] + mjs[jj]
            return 0

        lax.fori_loop(0, R_CHUNK, row_body, 0)


def _sc_add_one(x2, mod_emb, row):
    n = x2.shape[0]
    rows_w = n // NW
    chunks = rows_w // R_CHUNK
    full_iters = chunks // NBUF
    rem = chunks % NBUF
    assert full_iters >= 1
    mesh = plsc.VectorSubcoreMesh(core_axis_name="c", subcore_axis_name="s")

    @functools.partial(
        pl.kernel,
        mesh=mesh,
        out_type=jax.ShapeDtypeStruct((n, D), jnp.float32),
        scratch_types=(
            [pltpu.VMEM((R_CHUNK, D), jnp.float32)] * (2 * NBUF)
            + [pltpu.VMEM((D,), jnp.float32)]
            + [pltpu.SemaphoreType.DMA] * (2 * NBUF)
        ),
    )
    def k(x_hbm, mod_hbm, out_hbm, *scratch):
        ins = scratch[:NBUF]
        outs = scratch[NBUF:2 * NBUF]
        m0 = scratch[2 * NBUF]
        sins = scratch[2 * NBUF + 1:2 * NBUF + 1 + NBUF]
        souts = scratch[2 * NBUF + 1 + NBUF:]
        wid = lax.axis_index("s") * 2 + lax.axis_index("c")
        base = wid * rows_w
        pltpu.sync_copy(mod_hbm.at[row], m0)

        for b in range(NBUF):
            pltpu.async_copy(x_hbm.at[pl.ds(base + b * R_CHUNK, R_CHUNK)],
                             ins[b], sins[b])

        def do_chunk(c, b, maybe_first):
            off = base + c * R_CHUNK
            pltpu.make_async_copy(
                x_hbm.at[pl.ds(off, R_CHUNK)], ins[b], sins[b]).wait()

            def retire():
                pltpu.make_async_copy(
                    outs[b], out_hbm.at[pl.ds(off, R_CHUNK)], souts[b]).wait()
            if maybe_first is None:
                retire()
            else:
                pl.when(maybe_first > 0)(retire)

            _sc_compute_chunk(ins[b], outs[b], m0)
            def refill():
                pltpu.async_copy(
                    x_hbm.at[pl.ds(off + NBUF * R_CHUNK, R_CHUNK)],
                    ins[b], sins[b])
            if isinstance(c, int):
                if c + NBUF < chunks:
                    refill()
            else:
                pl.when(c + NBUF < chunks)(refill)
            pltpu.async_copy(outs[b], out_hbm.at[pl.ds(off, R_CHUNK)],
                             souts[b])

        def body(i, _):
            for b in range(NBUF):
                do_chunk(NBUF * i + b, b, i)
            return 0

        lax.fori_loop(0, full_iters, body, 0)
        for j in range(rem):
            do_chunk(NBUF * full_iters + j, j, None)
        for b in range(NBUF):
            pltpu.make_async_copy(
                outs[b], out_hbm.at[pl.ds(base, R_CHUNK)], souts[b]).wait()

    return k(x2, mod_emb)


def kernel(vision, language, mod_emb):
    b, lv, d = vision.shape
    _, lt, _ = language.shape
    ol = _tc_add_one(language.reshape(b * lt, d), mod_emb)
    ov = _sc_add_one(vision.reshape(b * lv, d), mod_emb, 0)
    return ov.reshape(b, lv, d), ol.reshape(b, lt, d)
